# baseline (speedup 1.0000x reference)
import functools

import jax
import jax.numpy as jnp
from jax import lax
from jax.experimental import pallas as pl
from jax.experimental.pallas import tpu as pltpu
from jax.experimental.pallas import tpu_sc as plsc

_MM_BLK = 8000
_LANES = 128


def _mm_body(x_ref, w_ref, o_ref):
    o_ref[...] = jnp.dot(x_ref[...], w_ref[...],
                         preferred_element_type=jnp.float32)


def _project_table(table, W):
    num, dim = table.shape
    w128 = jnp.concatenate(
        [W.T, jnp.zeros((dim, _LANES - W.shape[0]), jnp.float32)], axis=1)
    return pl.pallas_call(
        _mm_body,
        grid=(num // _MM_BLK,),
        in_specs=[
            pl.BlockSpec((_MM_BLK, dim), lambda i: (i, 0)),
            pl.BlockSpec((dim, _LANES), lambda i: (0, 0)),
        ],
        out_specs=pl.BlockSpec((_MM_BLK, _LANES), lambda i: (i, 0)),
        out_shape=jax.ShapeDtypeStruct((num, _LANES), jnp.float32),
    )(table, w128)


def _make_gather(b, l, dim, nb):
    info = plsc.get_sparse_core_info()
    nw = info.num_cores * info.num_subcores
    rows_w = b // nw
    n_chunks = rows_w // nb
    assert rows_w % nb == 0 and n_chunks % 2 == 0
    n_outer = n_chunks // 2
    mesh = plsc.VectorSubcoreMesh(core_axis_name="c", subcore_axis_name="s")

    @functools.partial(
        pl.kernel,
        mesh=mesh,
        out_type=jax.ShapeDtypeStruct((b, l, dim), jnp.float32),
        scratch_types=[
            pltpu.VMEM((nb, _LANES), jnp.int32),
            pltpu.VMEM((nb, _LANES), jnp.int32),
            pltpu.VMEM((nb * 56, _LANES), jnp.float32),
            pltpu.VMEM((nb * 56, _LANES), jnp.float32),
            pltpu.SemaphoreType.DMA,
            pltpu.SemaphoreType.DMA,
            pltpu.SemaphoreType.DMA,
            pltpu.SemaphoreType.DMA,
            pltpu.SemaphoreType.DMA,
            pltpu.SemaphoreType.DMA,
        ],
        compiler_params=pltpu.CompilerParams(use_tc_tiling_on_sc=False),
    )
    def gather(tab_hbm, idx_hbm, out_hbm, idxv0, idxv1, rows0, rows1,
               isem0, isem1, gsem0, gsem1, osem0, osem1):
        idxv = (idxv0, idxv1)
        rows = (rows0, rows1)
        isem = (isem0, isem1)
        gsem = (gsem0, gsem1)
        osem = (osem0, osem1)
        wid = lax.axis_index("s") * info.num_cores + lax.axis_index("c")
        row0 = wid * rows_w

        def stage_idx(g, b_):
            pltpu.async_copy(idx_hbm.at[pl.ds(row0 + g * nb, nb)], idxv[b_],
                             isem[b_])

        def wait_idx(g, b_):
            pltpu.make_async_copy(idx_hbm.at[pl.ds(row0 + g * nb, nb)],
                                  idxv[b_], isem[b_]).wait()

        def fire_gathers(b_):
            for k in range(nb):
                pltpu.async_copy(
                    tab_hbm.at[idxv[b_].at[k, pl.ds(0, 56)]],
                    rows[b_].at[pl.ds(k * 56, 56)], gsem[b_])

        def wait_gathers(b_):
            for k in range(nb):
                pltpu.make_async_copy(
                    tab_hbm.at[idxv[b_].at[k, pl.ds(0, 56)]],
                    rows[b_].at[pl.ds(k * 56, 56)], gsem[b_]).wait()

        def fire_puts(g, b_):
            for k in range(nb):
                pltpu.async_copy(
                    rows[b_].at[pl.ds(k * 56, l), pl.ds(0, dim)],
                    out_hbm.at[row0 + g * nb + k], osem[b_])

        def wait_puts(g, b_):
            for k in range(nb):
                pltpu.make_async_copy(
                    rows[b_].at[pl.ds(k * 56, l), pl.ds(0, dim)],
                    out_hbm.at[row0 + g * nb + k], osem[b_]).wait()

        stage_idx(0, 0)
        stage_idx(1, 1)

        def body(i, carry):
            g = i * 2
            wait_idx(g, 0)
            fire_gathers(0)
            wait_gathers(0)

            @pl.when(i + 1 < n_outer)
            def _():
                stage_idx(g + 2, 0)

            fire_puts(g, 0)
            wait_idx(g + 1, 1)
            fire_gathers(1)
            wait_gathers(1)

            @pl.when(i + 1 < n_outer)
            def _():
                stage_idx(g + 3, 1)

            fire_puts(g + 1, 1)
            wait_puts(g, 0)
            wait_puts(g + 1, 1)
            return carry

        lax.fori_loop(0, n_outer, body, 0)

    return gather


def kernel(indexes, table, W):
    b, l = indexes.shape
    num, dim = table.shape
    P128 = _project_table(table, W)
    idx128 = jnp.concatenate(
        [indexes.astype(jnp.int32),
         jnp.zeros((b, _LANES - l), jnp.int32)], axis=1)
    return _make_gather(b, l, dim, 8)(P128, idx128)


# speedup vs baseline: 3.7385x; 3.7385x over previous
import functools

import jax
import jax.numpy as jnp
from jax import lax
from jax.experimental import pallas as pl
from jax.experimental.pallas import tpu as pltpu
from jax.experimental.pallas import tpu_sc as plsc

_MM_BLK = 8000
_LANES = 128


def _mm_body(x_ref, w_ref, o_ref):
    o_ref[...] = jnp.dot(x_ref[...], w_ref[...],
                         preferred_element_type=jnp.float32)


def _project_table(table, W):
    num, dim = table.shape
    w128 = jnp.concatenate(
        [W.T, jnp.zeros((dim, _LANES - W.shape[0]), jnp.float32)], axis=1)
    return pl.pallas_call(
        _mm_body,
        grid=(num // _MM_BLK,),
        in_specs=[
            pl.BlockSpec((_MM_BLK, dim), lambda i: (i, 0)),
            pl.BlockSpec((dim, _LANES), lambda i: (0, 0)),
        ],
        out_specs=pl.BlockSpec((_MM_BLK, _LANES), lambda i: (i, 0)),
        out_shape=jax.ShapeDtypeStruct((num, _LANES), jnp.float32),
    )(table, w128)


def _make_gather(b, l, dim, nb):
    info = plsc.get_sparse_core_info()
    nw = info.num_cores * info.num_subcores
    rows_w = b // nw
    chunk = nb * l
    n_vec = chunk // 16
    n_chunks = rows_w // nb
    assert rows_w % nb == 0 and n_chunks % 2 == 0 and chunk % 16 == 0
    n_outer = n_chunks // 2
    mesh = plsc.VectorSubcoreMesh(core_axis_name="c", subcore_axis_name="s")

    @functools.partial(
        pl.kernel,
        mesh=mesh,
        out_type=jax.ShapeDtypeStruct((b, l, dim), jnp.float32),
        scratch_types=[
            pltpu.VMEM((8, _LANES), jnp.int32),
            pltpu.VMEM((8, _LANES), jnp.int32),
            pltpu.VMEM((nb, _LANES), jnp.int32),
            pltpu.VMEM((nb, _LANES), jnp.int32),
            pltpu.VMEM((chunk,), jnp.int32),
            pltpu.VMEM((chunk,), jnp.int32),
            pltpu.VMEM((chunk, _LANES), jnp.float32),
            pltpu.VMEM((chunk, _LANES), jnp.float32),
            pltpu.SemaphoreType.DMA,
            pltpu.SemaphoreType.DMA,
            pltpu.SemaphoreType.DMA,
            pltpu.SemaphoreType.DMA,
            pltpu.SemaphoreType.DMA,
            pltpu.SemaphoreType.DMA,
        ],
        compiler_params=pltpu.CompilerParams(use_tc_tiling_on_sc=False,
                                             needs_layout_passes=False),
    )
    def gather(tab_hbm, idx_hbm, rp_hbm, cp_hbm, out_hbm, rpv, cpv,
               idxv0, idxv1, list0, list1, rows0, rows1,
               isem0, isem1, gsem0, gsem1, osem0, osem1):
        pltpu.sync_copy(rp_hbm, rpv)
        pltpu.sync_copy(cp_hbm, cpv)
        idxv = (idxv0, idxv1)
        lists = (list0, list1)
        rows = (rows0, rows1)
        isem = (isem0, isem1)
        gsem = (gsem0, gsem1)
        osem = (osem0, osem1)
        wid = lax.axis_index("s") * info.num_cores + lax.axis_index("c")
        row0 = wid * rows_w

        def stage_idx(g, b_):
            pltpu.async_copy(idx_hbm.at[pl.ds(row0 + g * nb, nb)], idxv[b_],
                             isem[b_])

        def wait_idx(g, b_):
            pltpu.make_async_copy(idx_hbm.at[pl.ds(row0 + g * nb, nb)],
                                  idxv[b_], isem[b_]).wait()

        def compact(b_):
            for c in range(n_vec):
                p = c * 16
                rv = rpv[p // _LANES, pl.ds(p % _LANES, 16)]
                cv = cpv[p // _LANES, pl.ds(p % _LANES, 16)]
                lists[b_][pl.ds(c * 16, 16)] = plsc.load_gather(
                    idxv[b_], [rv, cv])

        def fire_gather(b_):
            pltpu.async_copy(tab_hbm.at[lists[b_]], rows[b_], gsem[b_])

        def wait_gather(b_):
            pltpu.make_async_copy(tab_hbm.at[lists[b_]], rows[b_],
                                  gsem[b_]).wait()

        def fire_puts(g, b_):
            for k in range(nb):
                pltpu.async_copy(
                    rows[b_].at[pl.ds(k * l, l), pl.ds(0, dim)],
                    out_hbm.at[row0 + g * nb + k], osem[b_])

        def wait_puts(g, b_):
            for k in range(nb):
                pltpu.make_async_copy(
                    rows[b_].at[pl.ds(k * l, l), pl.ds(0, dim)],
                    out_hbm.at[row0 + g * nb + k], osem[b_]).wait()

        stage_idx(0, 0)
        stage_idx(1, 1)

        def body(i, carry):
            g = i * 2
            wait_idx(g, 0)
            compact(0)
            fire_gather(0)
            wait_gather(0)

            @pl.when(i + 1 < n_outer)
            def _():
                stage_idx(g + 2, 0)

            fire_puts(g, 0)
            wait_idx(g + 1, 1)
            compact(1)
            fire_gather(1)
            wait_gather(1)

            @pl.when(i + 1 < n_outer)
            def _():
                stage_idx(g + 3, 1)

            fire_puts(g + 1, 1)
            wait_puts(g, 0)
            wait_puts(g + 1, 1)
            return carry

        lax.fori_loop(0, n_outer, body, 0)

    return gather


def kernel(indexes, table, W):
    b, l = indexes.shape
    num, dim = table.shape
    P128 = _project_table(table, W)
    idx128 = jnp.concatenate(
        [indexes.astype(jnp.int32),
         jnp.zeros((b, _LANES - l), jnp.int32)], axis=1)
    e = jnp.arange(8 * _LANES, dtype=jnp.int32)
    rp = (e // l).reshape(8, _LANES)
    cp = (e % l).reshape(8, _LANES)
    return _make_gather(b, l, dim, 8)(P128, idx128, rp, cp)
